# Initial kernel scaffold; baseline (speedup 1.0000x reference)
import jax
import jax.numpy as jnp
from jax.experimental import pallas as pl


def kernel(x, weight, bias):
    raise NotImplementedError("write your pallas kernel here")


# speedup vs baseline: 1.5499x; 1.5499x over previous
import functools

import jax
import jax.numpy as jnp
from jax.experimental import pallas as pl

T = 16384
D = 4096
E = 64
N_GROUPS = 8
G = E // N_GROUPS
TOPK_GROUPS = 4
TOPK = 8

BT = 512


def _router_body(x_ref, wt_ref, b_ref, w_out_ref, i_out_ref):
    scores = jnp.dot(x_ref[...], wt_ref[...], preferred_element_type=jnp.float32)
    m = jnp.max(scores, axis=-1, keepdims=True)
    e = jnp.exp(scores - m)
    probs = e / jnp.sum(e, axis=-1, keepdims=True)
    sb = probs + b_ref[...]

    lane_g = jax.lax.broadcasted_iota(jnp.int32, (1, N_GROUPS), 1)
    lane_e = jax.lax.broadcasted_iota(jnp.int32, (1, E), 1)
    group_of_lane = lane_e // G

    gparts = []
    lane_gi = jax.lax.broadcasted_iota(jnp.int32, (1, G), 1)
    for g in range(N_GROUPS):
        s_g = sb[:, g * G:(g + 1) * G]
        m1 = jnp.max(s_g, axis=-1, keepdims=True)
        idx1 = jnp.min(jnp.where(s_g == m1, lane_gi, G), axis=-1, keepdims=True)
        m2 = jnp.max(jnp.where(lane_gi == idx1, -jnp.inf, s_g), axis=-1, keepdims=True)
        gparts.append(m1 + m2)
    gs = jnp.concatenate(gparts, axis=-1)

    sel64 = jnp.zeros(sb.shape, dtype=jnp.bool_)
    gwork = gs
    for _ in range(TOPK_GROUPS):
        mg = jnp.max(gwork, axis=-1, keepdims=True)
        gidx = jnp.min(jnp.where(gwork == mg, lane_g, N_GROUPS), axis=-1, keepdims=True)
        gwork = jnp.where(lane_g == gidx, -jnp.inf, gwork)
        sel64 = sel64 | (group_of_lane == gidx)

    work = jnp.where(sel64, sb, -jnp.inf)

    idx_parts, val_parts = [], []
    for _ in range(TOPK):
        mv = jnp.max(work, axis=-1, keepdims=True)
        idx = jnp.min(jnp.where(work == mv, lane_e, E), axis=-1, keepdims=True)
        hit = lane_e == idx
        val_parts.append(jnp.sum(jnp.where(hit, probs, 0.0), axis=-1, keepdims=True))
        work = jnp.where(hit, -jnp.inf, work)
        idx_parts.append(idx)

    i_out_ref[...] = jnp.concatenate(idx_parts, axis=-1)
    w_out_ref[...] = jnp.concatenate(val_parts, axis=-1)


@functools.partial(jax.jit, static_argnames=())
def kernel(x, weight, bias):
    wt = weight.T
    b2 = bias.reshape(1, E)
    grid = (T // BT,)
    weights, indices = pl.pallas_call(
        _router_body,
        grid=grid,
        in_specs=[
            pl.BlockSpec((BT, D), lambda i: (i, 0)),
            pl.BlockSpec((D, E), lambda i: (0, 0)),
            pl.BlockSpec((1, E), lambda i: (0, 0)),
        ],
        out_specs=[
            pl.BlockSpec((BT, TOPK), lambda i: (i, 0)),
            pl.BlockSpec((BT, TOPK), lambda i: (i, 0)),
        ],
        out_shape=[
            jax.ShapeDtypeStruct((T, TOPK), jnp.float32),
            jax.ShapeDtypeStruct((T, TOPK), jnp.int32),
        ],
    )(x, wt, b2)
    return weights, indices


# speedup vs baseline: 4.9866x; 3.2174x over previous
import jax
import jax.numpy as jnp
from jax.experimental import pallas as pl
from jax.experimental.pallas import tpu as pltpu

T = 16384
D = 4096
E = 64
N_GROUPS = 8
G = E // N_GROUPS
TOPK_GROUPS = 4
TOPK = 8

BT = 512


def _router_body(x_ref, wt_ref, b_ref, w_out_ref, i_out_ref):
    s = jnp.dot(x_ref[...], wt_ref[...], preferred_element_type=jnp.float32)
    st = s.T

    m = jnp.max(st, axis=0, keepdims=True)
    e = jnp.exp(st - m)
    probs = e / jnp.sum(e, axis=0, keepdims=True)
    sb = probs + b_ref[...]

    iota_g = jax.lax.broadcasted_iota(jnp.int32, (G, 1), 0)
    iota_ng = jax.lax.broadcasted_iota(jnp.int32, (N_GROUPS, 1), 0)
    iota_e = jax.lax.broadcasted_iota(jnp.int32, (E, 1), 0)
    group_of_row = iota_e // G

    gparts = []
    for g in range(N_GROUPS):
        s_g = sb[g * G:(g + 1) * G, :]
        m1 = jnp.max(s_g, axis=0, keepdims=True)
        idx1 = jnp.min(jnp.where(s_g == m1, iota_g, G), axis=0, keepdims=True)
        m2 = jnp.max(jnp.where(iota_g == idx1, -jnp.inf, s_g), axis=0, keepdims=True)
        gparts.append(m1 + m2)
    gs = jnp.concatenate(gparts, axis=0)

    sel = jnp.zeros(sb.shape, dtype=jnp.bool_)
    gwork = gs
    for _ in range(TOPK_GROUPS):
        mg = jnp.max(gwork, axis=0, keepdims=True)
        gidx = jnp.min(jnp.where(gwork == mg, iota_ng, N_GROUPS), axis=0, keepdims=True)
        gwork = jnp.where(iota_ng == gidx, -jnp.inf, gwork)
        sel = sel | (group_of_row == gidx)

    work = jnp.where(sel, sb, -jnp.inf)

    idx_parts, val_parts = [], []
    for _ in range(TOPK):
        mv = jnp.max(work, axis=0, keepdims=True)
        idx = jnp.min(jnp.where(work == mv, iota_e, E), axis=0, keepdims=True)
        hit = iota_e == idx
        val_parts.append(jnp.sum(jnp.where(hit, probs, 0.0), axis=0, keepdims=True))
        work = jnp.where(hit, -jnp.inf, work)
        idx_parts.append(idx)

    ii = jnp.concatenate(idx_parts, axis=0)
    vv = jnp.concatenate(val_parts, axis=0)
    i_out_ref[...] = ii.T
    w_out_ref[...] = vv.T


def kernel(x, weight, bias):
    wt = weight.T
    b2 = bias.reshape(E, 1)
    grid = (T // BT,)
    weights, indices = pl.pallas_call(
        _router_body,
        grid=grid,
        in_specs=[
            pl.BlockSpec((BT, D), lambda i: (i, 0)),
            pl.BlockSpec((D, E), lambda i: (0, 0)),
            pl.BlockSpec((E, 1), lambda i: (0, 0)),
        ],
        out_specs=[
            pl.BlockSpec((BT, TOPK), lambda i: (i, 0)),
            pl.BlockSpec((BT, TOPK), lambda i: (i, 0)),
        ],
        out_shape=[
            jax.ShapeDtypeStruct((T, TOPK), jnp.float32),
            jax.ShapeDtypeStruct((T, TOPK), jnp.int32),
        ],
        compiler_params=pltpu.CompilerParams(
            dimension_semantics=("parallel",),
        ),
    )(x, wt, b2)
    return weights, indices


# speedup vs baseline: 5.3713x; 1.0772x over previous
import jax
import jax.numpy as jnp
from jax.experimental import pallas as pl
from jax.experimental.pallas import tpu as pltpu

T = 16384
D = 4096
E = 64
N_GROUPS = 8
G = E // N_GROUPS
TOPK_GROUPS = 4
TOPK = 8

BT = 1024


def _router_body(x_ref, wt_ref, b_ref, w_out_ref, i_out_ref):
    s = jnp.dot(x_ref[...], wt_ref[...], preferred_element_type=jnp.float32)
    st = s.T

    m = jnp.max(st, axis=0, keepdims=True)
    e = jnp.exp(st - m)
    probs = e / jnp.sum(e, axis=0, keepdims=True)
    sb = probs + b_ref[...]

    iota_g = jax.lax.broadcasted_iota(jnp.int32, (G, 1), 0)
    iota_ng = jax.lax.broadcasted_iota(jnp.int32, (N_GROUPS, 1), 0)
    iota_e = jax.lax.broadcasted_iota(jnp.int32, (E, 1), 0)
    group_of_row = iota_e // G

    gparts = []
    for g in range(N_GROUPS):
        s_g = sb[g * G:(g + 1) * G, :]
        m1 = jnp.max(s_g, axis=0, keepdims=True)
        idx1 = jnp.min(jnp.where(s_g == m1, iota_g, G), axis=0, keepdims=True)
        m2 = jnp.max(jnp.where(iota_g == idx1, -jnp.inf, s_g), axis=0, keepdims=True)
        gparts.append(m1 + m2)
    gs = jnp.concatenate(gparts, axis=0)

    sel = jnp.zeros(sb.shape, dtype=jnp.bool_)
    gwork = gs
    for _ in range(TOPK_GROUPS):
        mg = jnp.max(gwork, axis=0, keepdims=True)
        gidx = jnp.min(jnp.where(gwork == mg, iota_ng, N_GROUPS), axis=0, keepdims=True)
        gwork = jnp.where(iota_ng == gidx, -jnp.inf, gwork)
        sel = sel | (group_of_row == gidx)

    work = jnp.where(sel, sb, -jnp.inf)

    idx_parts, val_parts = [], []
    for _ in range(TOPK):
        mv = jnp.max(work, axis=0, keepdims=True)
        idx = jnp.min(jnp.where(work == mv, iota_e, E), axis=0, keepdims=True)
        hit = iota_e == idx
        val_parts.append(jnp.sum(jnp.where(hit, probs, 0.0), axis=0, keepdims=True))
        work = jnp.where(hit, -jnp.inf, work)
        idx_parts.append(idx)

    ii = jnp.concatenate(idx_parts, axis=0)
    vv = jnp.concatenate(val_parts, axis=0)
    i_out_ref[...] = ii.T
    w_out_ref[...] = vv.T


def kernel(x, weight, bias):
    wt = weight.T
    b2 = bias.reshape(E, 1)
    grid = (T // BT,)
    weights, indices = pl.pallas_call(
        _router_body,
        grid=grid,
        in_specs=[
            pl.BlockSpec((BT, D), lambda i: (i, 0)),
            pl.BlockSpec((D, E), lambda i: (0, 0)),
            pl.BlockSpec((E, 1), lambda i: (0, 0)),
        ],
        out_specs=[
            pl.BlockSpec((BT, TOPK), lambda i: (i, 0)),
            pl.BlockSpec((BT, TOPK), lambda i: (i, 0)),
        ],
        out_shape=[
            jax.ShapeDtypeStruct((T, TOPK), jnp.float32),
            jax.ShapeDtypeStruct((T, TOPK), jnp.int32),
        ],
        compiler_params=pltpu.CompilerParams(
            dimension_semantics=("parallel",),
        ),
    )(x, wt, b2)
    return weights, indices
